# baseline (speedup 1.0000x reference)
import functools

import jax
import jax.numpy as jnp
from jax import lax
from jax.experimental import pallas as pl
from jax.experimental.pallas import tpu as pltpu
from jax.experimental.pallas import tpu_sc as plsc

N = 320000
D = 128
K = 32
INTRA_MARGIN = 0.5
INTER_MARGIN = 1.5
INTRA_W = 1.0
INTER_W = 1.0
REG_W = 0.001

BN = 32000
NB = N // BN
CH = 8000
NCH = BN // CH
LCH = 2000

M = 256000
MB = M // BN
NC, NS = 2, 16
NW = NC * NS
RPT = (N - M) // NW
CHS = 400
NCHS = RPT // CHS


def _mm(a, b, dims):
    return jax.lax.dot_general(
        a, b, (dims, ((), ())), preferred_element_type=jnp.float32
    )


def _onehot_t(lab):
    return (
        lab == jax.lax.broadcasted_iota(lab.dtype, (K, 1), 0)
    ).astype(jnp.float32)


def _tc_phase0_kernel(lab_ref, f_ref, sums_ref, counts_ref):
    i = pl.program_id(0)

    @pl.when(i == 0)
    def _init():
        sums_ref[...] = jnp.zeros_like(sums_ref)
        counts_ref[...] = jnp.zeros_like(counts_ref)

    for j in range(BN // LCH):
        fs = f_ref[j * LCH:(j + 1) * LCH, :]
        oh_t = _onehot_t(lab_ref[j, :, :])
        sums_ref[...] += _mm(oh_t, fs, ((1,), (0,)))
        counts_ref[...] += _mm(
            oh_t, jnp.ones((LCH, 1), jnp.float32), ((1,), (0,))
        )


def _sc_phase0_kernel(f_hbm, lab_hbm, ones_hbm, zeros_hbm, zcnt_hbm,
                      sums_out, cnt_out, rows_v, lab_v, ones_v,
                      acc_sh, cnt_sh):
    cid = lax.axis_index("c")
    sid = lax.axis_index("s")
    wid = sid * NC + cid
    base = M + wid * RPT

    @pl.when(sid == 0)
    def _zero():
        pltpu.sync_copy(zeros_hbm, acc_sh)
        pltpu.sync_copy(zcnt_hbm, cnt_sh)

    pltpu.sync_copy(ones_hbm, ones_v)
    plsc.subcore_barrier()

    for ci in range(NCHS):
        pltpu.sync_copy(f_hbm.at[pl.ds(base + ci * CHS, CHS)], rows_v)
        pltpu.sync_copy(lab_hbm.at[pl.ds(base + ci * CHS, CHS)], lab_v)
        pltpu.sync_copy(rows_v, acc_sh.at[lab_v], add=True)
        pltpu.sync_copy(ones_v, cnt_sh.at[lab_v], add=True)

    plsc.subcore_barrier()

    @pl.when(sid == 0)
    def _flush():
        pltpu.sync_copy(acc_sh, sums_out.at[cid])
        pltpu.sync_copy(cnt_sh, cnt_out.at[cid])


def _tc_phase1_kernel(lab_ref, f_ref, tcs_ref, tcc_ref, scs_ref, scc_ref,
                      out_ref, means_ref, counts_ref, intra_ref):
    i = pl.program_id(0)

    @pl.when(i == 0)
    def _means():
        sums = tcs_ref[...] + scs_ref[0] + scs_ref[1]
        counts = tcc_ref[...] + scc_ref[0, :, 0:1] + scc_ref[1, :, 0:1]
        counts_ref[...] = counts
        means_ref[...] = sums / counts
        intra_ref[...] = jnp.zeros_like(intra_ref)

    means = means_ref[...]
    c = means - 1e-08
    csq_row = jnp.sum(c * c, axis=1)[None, :]
    for j in range(NCH):
        f = f_ref[j * CH:(j + 1) * CH, :]
        oh_t = _onehot_t(lab_ref[j, :, :])
        f2 = f * f
        dots_t = _mm(c, f, ((1,), (1,)))
        q_t = _mm(jnp.ones((1, D), jnp.float32), f2, ((1,), (1,)))
        seldot = _mm(
            jnp.ones((1, K), jnp.float32), oh_t * dots_t, ((1,), (0,))
        )
        selcsq = _mm(csq_row, oh_t, ((1,), (0,)))
        dist2 = q_t - 2.0 * seldot + selcsq
        dist = jnp.sqrt(dist2)
        hinge = jnp.maximum(dist - INTRA_MARGIN, 0.0)
        h2m_t = oh_t * (hinge * hinge)
        intra_ref[...] += _mm(
            h2m_t, jnp.ones((CH, 1), jnp.float32), ((1,), (0,))
        )

    @pl.when(i == NB - 1)
    def _finish():
        intra_loss = jnp.sum(intra_ref[:, 0] / counts_ref[:, 0]) / K

        md = means[:, None, :] - means[None, :, :] + 1e-08
        pair_dist = jnp.sqrt(jnp.sum(md * md, axis=-1))
        pair_hinge = jnp.maximum(2.0 * INTER_MARGIN - pair_dist, 0.0)
        offdiag = 1.0 - jnp.eye(K, dtype=jnp.float32)
        inter_loss = jnp.sum(pair_hinge * pair_hinge * offdiag) / float(
            (K - 1) * K
        )

        mr = means + 1e-08
        reg_loss = jnp.sum(jnp.sqrt(jnp.sum(mr * mr, axis=1))) / float(K)

        loss = INTRA_W * intra_loss + INTER_W * inter_loss + REG_W * reg_loss
        out_ref[...] = jnp.broadcast_to(loss, (1, 1))


_sc_phase0 = functools.partial(
    pl.kernel,
    out_type=(
        jax.ShapeDtypeStruct((NC, K, D), jnp.float32),
        jax.ShapeDtypeStruct((NC, K, 16), jnp.float32),
    ),
    mesh=plsc.VectorSubcoreMesh(core_axis_name="c", subcore_axis_name="s"),
    scratch_types=[
        pltpu.VMEM((CHS, D), jnp.float32),
        pltpu.VMEM((CHS,), jnp.int32),
        pltpu.VMEM((CHS, 16), jnp.float32),
        pltpu.VMEM_SHARED((K, D), jnp.float32),
        pltpu.VMEM_SHARED((K, 16), jnp.float32),
    ],
)(_sc_phase0_kernel)


@jax.jit
def kernel(features, labels):
    labels_i32 = labels.astype(jnp.int32)
    labels3 = labels_i32.reshape(N // LCH, 1, LCH)

    tc_sums, tc_counts = pl.pallas_call(
        _tc_phase0_kernel,
        grid=(MB,),
        in_specs=[
            pl.BlockSpec((BN // LCH, 1, LCH), lambda i: (i, 0, 0)),
            pl.BlockSpec((BN, D), lambda i: (i, 0)),
        ],
        out_specs=[
            pl.BlockSpec((K, D), lambda i: (0, 0)),
            pl.BlockSpec((K, 1), lambda i: (0, 0)),
        ],
        out_shape=[
            jax.ShapeDtypeStruct((K, D), jnp.float32),
            jax.ShapeDtypeStruct((K, 1), jnp.float32),
        ],
    )(labels3, features)

    sc_sums, sc_counts = _sc_phase0(
        features,
        labels_i32,
        jnp.ones((CHS, 16), jnp.float32),
        jnp.zeros((K, D), jnp.float32),
        jnp.zeros((K, 16), jnp.float32),
    )

    labels3b = labels_i32.reshape(N // CH, 1, CH)
    out = pl.pallas_call(
        _tc_phase1_kernel,
        grid=(NB,),
        in_specs=[
            pl.BlockSpec((NCH, 1, CH), lambda i: (i, 0, 0)),
            pl.BlockSpec((BN, D), lambda i: (i, 0)),
            pl.BlockSpec((K, D), lambda i: (0, 0)),
            pl.BlockSpec((K, 1), lambda i: (0, 0)),
            pl.BlockSpec((NC, K, D), lambda i: (0, 0, 0)),
            pl.BlockSpec((NC, K, 16), lambda i: (0, 0, 0)),
        ],
        out_specs=pl.BlockSpec((1, 1), lambda i: (0, 0)),
        out_shape=jax.ShapeDtypeStruct((1, 1), jnp.float32),
        scratch_shapes=[
            pltpu.VMEM((K, D), jnp.float32),
            pltpu.VMEM((K, 1), jnp.float32),
            pltpu.VMEM((K, 1), jnp.float32),
        ],
    )(labels3b, features, tc_sums, tc_counts, sc_sums, sc_counts)
    return out.reshape(())


# speedup vs baseline: 1.0155x; 1.0155x over previous
import functools

import jax
import jax.numpy as jnp
from jax import lax
from jax.experimental import pallas as pl
from jax.experimental.pallas import tpu as pltpu
from jax.experimental.pallas import tpu_sc as plsc

N = 320000
D = 128
K = 32
INTRA_MARGIN = 0.5
INTER_MARGIN = 1.5
INTRA_W = 1.0
INTER_W = 1.0
REG_W = 0.001

BN = 32000
NB = N // BN
CH = 8000
NCH = BN // CH
LCH = 2000

M = 288000
MB = M // BN
NC, NS = 2, 16
NW = NC * NS
RPT = (N - M) // NW
CHS = 200
NCHS = RPT // CHS


def _mm(a, b, dims):
    return jax.lax.dot_general(
        a, b, (dims, ((), ())), preferred_element_type=jnp.float32
    )


def _onehot_t(lab):
    return (
        lab == jax.lax.broadcasted_iota(lab.dtype, (K, 1), 0)
    ).astype(jnp.float32)


def _tc_phase0_kernel(lab_ref, f_ref, sums_ref, counts_ref):
    i = pl.program_id(0)

    @pl.when(i == 0)
    def _init():
        sums_ref[...] = jnp.zeros_like(sums_ref)
        counts_ref[...] = jnp.zeros_like(counts_ref)

    for j in range(BN // LCH):
        fs = f_ref[j * LCH:(j + 1) * LCH, :]
        oh_t = _onehot_t(lab_ref[j, :, :])
        sums_ref[...] += _mm(oh_t, fs, ((1,), (0,)))
        counts_ref[...] += _mm(
            oh_t, jnp.ones((LCH, 1), jnp.float32), ((1,), (0,))
        )


def _sc_phase0_kernel(f_hbm, lab_hbm, ones_hbm, zeros_hbm, zcnt_hbm,
                      sums_out, cnt_out, rows_v, lab_v, ones_v,
                      acc_sh, cnt_sh):
    cid = lax.axis_index("c")
    sid = lax.axis_index("s")
    wid = sid * NC + cid
    base = M + wid * RPT

    @pl.when(sid == 0)
    def _zero():
        pltpu.sync_copy(zeros_hbm, acc_sh)
        pltpu.sync_copy(zcnt_hbm, cnt_sh)

    pltpu.sync_copy(ones_hbm, ones_v)
    plsc.subcore_barrier()

    for ci in range(NCHS):
        pltpu.sync_copy(f_hbm.at[pl.ds(base + ci * CHS, CHS)], rows_v)
        pltpu.sync_copy(lab_hbm.at[pl.ds(base + ci * CHS, CHS)], lab_v)
        pltpu.sync_copy(rows_v, acc_sh.at[lab_v], add=True)
        pltpu.sync_copy(ones_v, cnt_sh.at[lab_v], add=True)

    plsc.subcore_barrier()

    @pl.when(sid == 0)
    def _flush():
        pltpu.sync_copy(acc_sh, sums_out.at[cid])
        pltpu.sync_copy(cnt_sh, cnt_out.at[cid])


def _tc_phase1_kernel(lab_ref, f_ref, tcs_ref, tcc_ref, scs_ref, scc_ref,
                      out_ref, means_ref, counts_ref, intra_ref):
    i = pl.program_id(0)

    @pl.when(i == 0)
    def _means():
        sums = tcs_ref[...] + scs_ref[0] + scs_ref[1]
        counts = tcc_ref[...] + scc_ref[0, :, 0:1] + scc_ref[1, :, 0:1]
        counts_ref[...] = counts
        means_ref[...] = sums / counts
        intra_ref[...] = jnp.zeros_like(intra_ref)

    means = means_ref[...]
    c = means - 1e-08
    csq_row = jnp.sum(c * c, axis=1)[None, :]
    for j in range(NCH):
        f = f_ref[j * CH:(j + 1) * CH, :]
        oh_t = _onehot_t(lab_ref[j, :, :])
        f2 = f * f
        dots_t = _mm(c, f, ((1,), (1,)))
        q_t = _mm(jnp.ones((1, D), jnp.float32), f2, ((1,), (1,)))
        seldot = _mm(
            jnp.ones((1, K), jnp.float32), oh_t * dots_t, ((1,), (0,))
        )
        selcsq = _mm(csq_row, oh_t, ((1,), (0,)))
        dist2 = q_t - 2.0 * seldot + selcsq
        dist = jnp.sqrt(dist2)
        hinge = jnp.maximum(dist - INTRA_MARGIN, 0.0)
        h2m_t = oh_t * (hinge * hinge)
        intra_ref[...] += _mm(
            h2m_t, jnp.ones((CH, 1), jnp.float32), ((1,), (0,))
        )

    @pl.when(i == NB - 1)
    def _finish():
        intra_loss = jnp.sum(intra_ref[:, 0] / counts_ref[:, 0]) / K

        md = means[:, None, :] - means[None, :, :] + 1e-08
        pair_dist = jnp.sqrt(jnp.sum(md * md, axis=-1))
        pair_hinge = jnp.maximum(2.0 * INTER_MARGIN - pair_dist, 0.0)
        offdiag = 1.0 - jnp.eye(K, dtype=jnp.float32)
        inter_loss = jnp.sum(pair_hinge * pair_hinge * offdiag) / float(
            (K - 1) * K
        )

        mr = means + 1e-08
        reg_loss = jnp.sum(jnp.sqrt(jnp.sum(mr * mr, axis=1))) / float(K)

        loss = INTRA_W * intra_loss + INTER_W * inter_loss + REG_W * reg_loss
        out_ref[...] = jnp.broadcast_to(loss, (1, 1))


_sc_phase0 = functools.partial(
    pl.kernel,
    out_type=(
        jax.ShapeDtypeStruct((NC, K, D), jnp.float32),
        jax.ShapeDtypeStruct((NC, K, 16), jnp.float32),
    ),
    mesh=plsc.VectorSubcoreMesh(core_axis_name="c", subcore_axis_name="s"),
    scratch_types=[
        pltpu.VMEM((CHS, D), jnp.float32),
        pltpu.VMEM((CHS,), jnp.int32),
        pltpu.VMEM((CHS, 16), jnp.float32),
        pltpu.VMEM_SHARED((K, D), jnp.float32),
        pltpu.VMEM_SHARED((K, 16), jnp.float32),
    ],
)(_sc_phase0_kernel)


@jax.jit
def kernel(features, labels):
    labels_i32 = labels.astype(jnp.int32)
    labels3 = labels_i32.reshape(N // LCH, 1, LCH)

    tc_sums, tc_counts = pl.pallas_call(
        _tc_phase0_kernel,
        grid=(MB,),
        in_specs=[
            pl.BlockSpec((BN // LCH, 1, LCH), lambda i: (i, 0, 0)),
            pl.BlockSpec((BN, D), lambda i: (i, 0)),
        ],
        out_specs=[
            pl.BlockSpec((K, D), lambda i: (0, 0)),
            pl.BlockSpec((K, 1), lambda i: (0, 0)),
        ],
        out_shape=[
            jax.ShapeDtypeStruct((K, D), jnp.float32),
            jax.ShapeDtypeStruct((K, 1), jnp.float32),
        ],
    )(labels3, features)

    sc_sums, sc_counts = _sc_phase0(
        features,
        labels_i32,
        jnp.ones((CHS, 16), jnp.float32),
        jnp.zeros((K, D), jnp.float32),
        jnp.zeros((K, 16), jnp.float32),
    )

    labels3b = labels_i32.reshape(N // CH, 1, CH)
    out = pl.pallas_call(
        _tc_phase1_kernel,
        grid=(NB,),
        in_specs=[
            pl.BlockSpec((NCH, 1, CH), lambda i: (i, 0, 0)),
            pl.BlockSpec((BN, D), lambda i: (i, 0)),
            pl.BlockSpec((K, D), lambda i: (0, 0)),
            pl.BlockSpec((K, 1), lambda i: (0, 0)),
            pl.BlockSpec((NC, K, D), lambda i: (0, 0, 0)),
            pl.BlockSpec((NC, K, 16), lambda i: (0, 0, 0)),
        ],
        out_specs=pl.BlockSpec((1, 1), lambda i: (0, 0)),
        out_shape=jax.ShapeDtypeStruct((1, 1), jnp.float32),
        scratch_shapes=[
            pltpu.VMEM((K, D), jnp.float32),
            pltpu.VMEM((K, 1), jnp.float32),
            pltpu.VMEM((K, 1), jnp.float32),
        ],
    )(labels3b, features, tc_sums, tc_counts, sc_sums, sc_counts)
    return out.reshape(())


# speedup vs baseline: 1.0173x; 1.0018x over previous
import functools

import jax
import jax.numpy as jnp
from jax import lax
from jax.experimental import pallas as pl
from jax.experimental.pallas import tpu as pltpu
from jax.experimental.pallas import tpu_sc as plsc

N = 320000
D = 128
K = 32
INTRA_MARGIN = 0.5
INTER_MARGIN = 1.5
INTRA_W = 1.0
INTER_W = 1.0
REG_W = 0.001

BN = 32000
NB = N // BN
CH = 8000
NCH = BN // CH
LCH = 2000

M = 288000
BN1 = 32000
MB = M // BN1
NC, NS = 2, 16
NW = NC * NS
RPT = (N - M) // NW
CHS = 200
NCHS = RPT // CHS


def _mm(a, b, dims):
    return jax.lax.dot_general(
        a, b, (dims, ((), ())), preferred_element_type=jnp.float32
    )


def _onehot_t(lab):
    return (
        lab == jax.lax.broadcasted_iota(lab.dtype, (K, 1), 0)
    ).astype(jnp.float32)


def _tc_phase0_kernel(lab_ref, f_ref, sums_ref, counts_ref):
    i = pl.program_id(0)

    @pl.when(i == 0)
    def _init():
        sums_ref[...] = jnp.zeros_like(sums_ref)
        counts_ref[...] = jnp.zeros_like(counts_ref)

    for j in range(BN1 // LCH):
        fs = f_ref[j * LCH:(j + 1) * LCH, :]
        oh_t = _onehot_t(lab_ref[j, :, :])
        sums_ref[...] += _mm(oh_t, fs, ((1,), (0,)))
        counts_ref[...] += _mm(
            oh_t, jnp.ones((LCH, 1), jnp.float32), ((1,), (0,))
        )


def _sc_phase0_kernel(f_hbm, lab_hbm, ones_hbm, zeros_hbm, zcnt_hbm,
                      sums_out, cnt_out, rows_v, lab_v, ones_v,
                      acc_sh, cnt_sh):
    cid = lax.axis_index("c")
    sid = lax.axis_index("s")
    wid = sid * NC + cid
    base = M + wid * RPT

    @pl.when(sid == 0)
    def _zero():
        pltpu.sync_copy(zeros_hbm, acc_sh)
        pltpu.sync_copy(zcnt_hbm, cnt_sh)

    pltpu.sync_copy(ones_hbm, ones_v)
    plsc.subcore_barrier()

    for ci in range(NCHS):
        pltpu.sync_copy(f_hbm.at[pl.ds(base + ci * CHS, CHS)], rows_v)
        pltpu.sync_copy(lab_hbm.at[pl.ds(base + ci * CHS, CHS)], lab_v)
        pltpu.sync_copy(rows_v, acc_sh.at[lab_v], add=True)
        pltpu.sync_copy(ones_v, cnt_sh.at[lab_v], add=True)

    plsc.subcore_barrier()

    @pl.when(sid == 0)
    def _flush():
        pltpu.sync_copy(acc_sh, sums_out.at[cid])
        pltpu.sync_copy(cnt_sh, cnt_out.at[cid])


def _tc_phase1_kernel(lab_ref, f_ref, tcs_ref, tcc_ref, scs_ref, scc_ref,
                      out_ref, means_ref, counts_ref, intra_ref):
    i = pl.program_id(0)

    @pl.when(i == 0)
    def _means():
        sums = tcs_ref[...] + scs_ref[0] + scs_ref[1]
        counts = tcc_ref[...] + scc_ref[0, :, 0:1] + scc_ref[1, :, 0:1]
        counts_ref[...] = counts
        means_ref[...] = sums / counts
        intra_ref[...] = jnp.zeros_like(intra_ref)

    means = means_ref[...]
    c = means - 1e-08
    csq_row = jnp.sum(c * c, axis=1)[None, :]
    for j in range(NCH):
        f = f_ref[j * CH:(j + 1) * CH, :]
        oh_t = _onehot_t(lab_ref[j, :, :])
        f2 = f * f
        dots_t = _mm(c, f, ((1,), (1,)))
        q_t = _mm(jnp.ones((1, D), jnp.float32), f2, ((1,), (1,)))
        seldot = _mm(
            jnp.ones((1, K), jnp.float32), oh_t * dots_t, ((1,), (0,))
        )
        selcsq = _mm(csq_row, oh_t, ((1,), (0,)))
        dist2 = q_t - 2.0 * seldot + selcsq
        dist = jnp.sqrt(dist2)
        hinge = jnp.maximum(dist - INTRA_MARGIN, 0.0)
        h2m_t = oh_t * (hinge * hinge)
        intra_ref[...] += _mm(
            h2m_t, jnp.ones((CH, 1), jnp.float32), ((1,), (0,))
        )

    @pl.when(i == NB - 1)
    def _finish():
        intra_loss = jnp.sum(intra_ref[:, 0] / counts_ref[:, 0]) / K

        md = means[:, None, :] - means[None, :, :] + 1e-08
        pair_dist = jnp.sqrt(jnp.sum(md * md, axis=-1))
        pair_hinge = jnp.maximum(2.0 * INTER_MARGIN - pair_dist, 0.0)
        offdiag = 1.0 - jnp.eye(K, dtype=jnp.float32)
        inter_loss = jnp.sum(pair_hinge * pair_hinge * offdiag) / float(
            (K - 1) * K
        )

        mr = means + 1e-08
        reg_loss = jnp.sum(jnp.sqrt(jnp.sum(mr * mr, axis=1))) / float(K)

        loss = INTRA_W * intra_loss + INTER_W * inter_loss + REG_W * reg_loss
        out_ref[...] = jnp.broadcast_to(loss, (1, 1))


_sc_phase0 = functools.partial(
    pl.kernel,
    out_type=(
        jax.ShapeDtypeStruct((NC, K, D), jnp.float32),
        jax.ShapeDtypeStruct((NC, K, 16), jnp.float32),
    ),
    mesh=plsc.VectorSubcoreMesh(core_axis_name="c", subcore_axis_name="s"),
    scratch_types=[
        pltpu.VMEM((CHS, D), jnp.float32),
        pltpu.VMEM((CHS,), jnp.int32),
        pltpu.VMEM((CHS, 16), jnp.float32),
        pltpu.VMEM_SHARED((K, D), jnp.float32),
        pltpu.VMEM_SHARED((K, 16), jnp.float32),
    ],
)(_sc_phase0_kernel)


@jax.jit
def kernel(features, labels):
    labels_i32 = labels.astype(jnp.int32)
    labels3 = labels_i32.reshape(N // LCH, 1, LCH)

    tc_sums, tc_counts = pl.pallas_call(
        _tc_phase0_kernel,
        grid=(MB,),
        in_specs=[
            pl.BlockSpec((BN1 // LCH, 1, LCH), lambda i: (i, 0, 0)),
            pl.BlockSpec((BN1, D), lambda i: (i, 0)),
        ],
        out_specs=[
            pl.BlockSpec((K, D), lambda i: (0, 0)),
            pl.BlockSpec((K, 1), lambda i: (0, 0)),
        ],
        out_shape=[
            jax.ShapeDtypeStruct((K, D), jnp.float32),
            jax.ShapeDtypeStruct((K, 1), jnp.float32),
        ],
    )(labels3, features)

    sc_sums, sc_counts = _sc_phase0(
        features,
        labels_i32,
        jnp.ones((CHS, 16), jnp.float32),
        jnp.zeros((K, D), jnp.float32),
        jnp.zeros((K, 16), jnp.float32),
    )

    labels3b = labels_i32.reshape(N // CH, 1, CH)
    out = pl.pallas_call(
        _tc_phase1_kernel,
        grid=(NB,),
        in_specs=[
            pl.BlockSpec((NCH, 1, CH), lambda i: (i, 0, 0)),
            pl.BlockSpec((BN, D), lambda i: (i, 0)),
            pl.BlockSpec((K, D), lambda i: (0, 0)),
            pl.BlockSpec((K, 1), lambda i: (0, 0)),
            pl.BlockSpec((NC, K, D), lambda i: (0, 0, 0)),
            pl.BlockSpec((NC, K, 16), lambda i: (0, 0, 0)),
        ],
        out_specs=pl.BlockSpec((1, 1), lambda i: (0, 0)),
        out_shape=jax.ShapeDtypeStruct((1, 1), jnp.float32),
        scratch_shapes=[
            pltpu.VMEM((K, D), jnp.float32),
            pltpu.VMEM((K, 1), jnp.float32),
            pltpu.VMEM((K, 1), jnp.float32),
        ],
    )(labels3b, features, tc_sums, tc_counts, sc_sums, sc_counts)
    return out.reshape(())
